# baseline (speedup 1.0000x reference)
import functools

import jax
import jax.numpy as jnp
from jax import lax
from jax.experimental import pallas as pl
from jax.experimental.pallas import tpu as pltpu
from jax.experimental.pallas import tpu_sc as plsc

NUM_WORKERS = 32
BGRP = 128
HTILE = 8
RING = 4


@functools.lru_cache(maxsize=None)
def _make_gather(hist: int, batch: int, vocab: int, embed: int):
    assert embed == 32 and vocab % 4 == 0
    assert batch % (NUM_WORKERS * BGRP) == 0 and hist % HTILE == 0
    n_sub = batch // (NUM_WORKERS * BGRP)
    n_ht = hist // HTILE
    mesh = plsc.VectorSubcoreMesh(core_axis_name="c", subcore_axis_name="s")

    @functools.partial(
        pl.kernel,
        mesh=mesh,
        out_type=jax.ShapeDtypeStruct((hist, embed, batch), jnp.float32),
        scratch_types=[
            pltpu.VMEM((2, HTILE, BGRP), jnp.int32),
            pltpu.VMEM((RING, BGRP), jnp.int32),
            pltpu.VMEM((RING, BGRP, 128), jnp.float32),
            pltpu.VMEM((2, embed, BGRP), jnp.float32),
            [pltpu.SemaphoreType.DMA] * RING,
            [pltpu.SemaphoreType.DMA] * 2,
            pltpu.SemaphoreType.DMA,
        ],
        compiler_params=pltpu.CompilerParams(
            use_tc_tiling_on_sc=True, needs_layout_passes=False),
    )
    def gather_kernel(idx_hbm, table_hbm, out_hbm, idx_v, j_v, rows_v, blk_v,
                      gsems, ssems, isem):
        wid = lax.axis_index("s") * 2 + lax.axis_index("c")
        iota = lax.iota(jnp.int32, 16)

        def load_idx_tile(ht, tb, boff, sem):
            return pltpu.async_copy(
                idx_hbm.at[pl.ds(ht * HTILE, HTILE), pl.ds(boff, BGRP)],
                idx_v.at[tb], sem)

        def fire_gather(tb, hrow, s):
            for i in range(BGRP // 16):
                v = idx_v[tb, hrow, pl.ds(i * 16, 16)]
                j_v[s, pl.ds(i * 16, 16)] = lax.shift_right_logical(v, 2)
            pltpu.async_copy(table_hbm.at[j_v.at[s]], rows_v.at[s], gsems[s])

        def wait_gather(s):
            pltpu.make_async_copy(
                table_hbm.at[j_v.at[s]], rows_v.at[s], gsems[s]).wait()

        def transpose(tb, hr, s, s2):
            cbv = []
            rowv = []
            for bg in range(BGRP // 16):
                v = idx_v[tb, hr, pl.ds(bg * 16, 16)]
                cbv.append((v & 3) << 5)
                rowv.append(bg * 16 + iota)

            @plsc.parallel_loop(0, embed, unroll=4)
            def _(e):
                for bg in range(BGRP // 16):
                    g = plsc.load_gather(rows_v.at[s], [rowv[bg], cbv[bg] | e])
                    blk_v[s2, e, pl.ds(bg * 16, 16)] = g

        def fire_store(h, boff, s2):
            pltpu.async_copy(
                blk_v.at[s2], out_hbm.at[h, pl.ds(0, embed), pl.ds(boff, BGRP)],
                ssems[s2])

        def wait_store(h, boff, s2):
            pltpu.make_async_copy(
                blk_v.at[s2], out_hbm.at[h, pl.ds(0, embed), pl.ds(boff, BGRP)],
                ssems[s2]).wait()

        for sub in range(n_sub):
            boff = wid * (n_sub * BGRP) + sub * BGRP

            load_idx_tile(0, 0, boff, isem).wait()
            for t in range(RING):
                fire_gather(0, t, t)
            load_idx_tile(1, 1, boff, isem)

            @pl.loop(0, n_ht)
            def _(ht):
                tbt = ht & 1
                h0 = ht * HTILE
                for hr in range(HTILE):
                    s = hr & (RING - 1)
                    s2 = hr & 1
                    if hr == 0:
                        @pl.when(jnp.logical_and(ht >= 1, ht < n_ht - 1))
                        def _():
                            load_idx_tile(ht + 1, 1 - tbt, boff, isem)
                    wait_gather(s)
                    if hr < 2:
                        @pl.when(ht > 0)
                        def _():
                            wait_store(h0 + hr, boff, s2)
                    else:
                        wait_store(h0 + hr, boff, s2)
                    transpose(tbt, hr, s, s2)
                    fire_store(h0 + hr, boff, s2)
                    if hr < HTILE - RING:
                        fire_gather(tbt, hr + RING, s)
                    else:
                        if hr == HTILE - RING:
                            @pl.when(ht < n_ht - 1)
                            def _():
                                pltpu.make_async_copy(
                                    idx_hbm.at[pl.ds((ht + 1) * HTILE, HTILE),
                                               pl.ds(boff, BGRP)],
                                    idx_v.at[1 - tbt], isem).wait()

                        @pl.when(ht < n_ht - 1)
                        def _():
                            fire_gather(1 - tbt, hr - (HTILE - RING), s)

            wait_store(hist - 2, boff, 0)
            wait_store(hist - 1, boff, 1)

    return gather_kernel


def kernel(input, table):
    batch, hist = input.shape
    vocab, embed = table.shape
    idx_t = jnp.transpose(input.astype(jnp.int32))
    table128 = table.reshape(vocab // 4, 4 * embed)
    out_phys = _make_gather(hist, batch, vocab, embed)(idx_t, table128)
    return jnp.transpose(out_phys, (2, 0, 1))


# speedup vs baseline: 1.0075x; 1.0075x over previous
import functools

import jax
import jax.numpy as jnp
from jax import lax
from jax.experimental import pallas as pl
from jax.experimental.pallas import tpu as pltpu
from jax.experimental.pallas import tpu_sc as plsc

NUM_WORKERS = 32
HTILE = 8
G1 = 512
RING = 4


@functools.lru_cache(maxsize=None)
def _make_kernel(hist: int, batch: int, vocab: int, embed: int):
    assert batch % (NUM_WORKERS * G1) == 0 and hist % HTILE == 0
    assert embed % 8 == 0 and batch % 128 == 0
    n_sub = batch // (NUM_WORKERS * G1)
    n_ht = hist // HTILE
    nbt = G1 // 128
    mesh = plsc.VectorSubcoreMesh(core_axis_name="c", subcore_axis_name="s")

    @functools.partial(
        pl.kernel,
        mesh=mesh,
        out_type=jax.ShapeDtypeStruct(
            (hist, embed // 8, batch // 128, 8, 128), jnp.float32),
        scratch_types=[
            pltpu.VMEM((2, HTILE, G1), jnp.int32),
            pltpu.VMEM((RING, G1 // 128, 128), jnp.int32),
            pltpu.VMEM((RING, G1, embed), jnp.float32),
            pltpu.VMEM((2, embed // 8, 8, 128), jnp.float32),
            [pltpu.SemaphoreType.DMA] * RING,
            [pltpu.SemaphoreType.DMA] * 2,
            pltpu.SemaphoreType.DMA,
        ],
        compiler_params=pltpu.CompilerParams(
            use_tc_tiling_on_sc=False, needs_layout_passes=False),
    )
    def k(idx_hbm, table_hbm, out_hbm, idx_v, j_v, rows_v, blk_v, gsems,
          ssems, isem):
        wid = lax.axis_index("s") * 2 + lax.axis_index("c")
        iota = lax.iota(jnp.int32, 16)
        zeros16 = iota >> 4

        def load_idx_tile(ht, tb, boff, sem):
            return pltpu.async_copy(
                idx_hbm.at[pl.ds(ht * HTILE, HTILE), pl.ds(boff, G1)],
                idx_v.at[tb], sem)

        def fire_gather(tb, hrow, s):
            for kk in range(G1 // 128):
                for i in range(8):
                    v = idx_v[tb, hrow, pl.ds(kk * 128 + i * 16, 16)]
                    j_v[s, kk, pl.ds(i * 16, 16)] = v
            for kk in range(G1 // 128):
                pltpu.async_copy(
                    table_hbm.at[j_v.at[s, kk]],
                    rows_v.at[s, pl.ds(kk * 128, 128)], gsems[s])

        def wait_gather(s):
            for kk in range(G1 // 128):
                pltpu.make_async_copy(
                    table_hbm.at[j_v.at[s, kk]],
                    rows_v.at[s, pl.ds(kk * 128, 128)], gsems[s]).wait()

        def transpose(s, bt, s2):
            @plsc.parallel_loop(0, embed, unroll=4)
            def _(e):
                for bg in range(8):
                    row = bt * 128 + bg * 16 + iota
                    g = plsc.load_gather(rows_v.at[s], [row, zeros16 + e])
                    blk_v[s2, e >> 3, e & 7, pl.ds(bg * 16, 16)] = g

        def fire_store(h, btg, s2):
            pltpu.async_copy(
                blk_v.at[s2], out_hbm.at[h, pl.ds(0, embed // 8), btg],
                ssems[s2])

        def wait_store(h, btg, s2):
            pltpu.make_async_copy(
                blk_v.at[s2], out_hbm.at[h, pl.ds(0, embed // 8), btg],
                ssems[s2]).wait()

        for sub in range(n_sub):
            boff = wid * (n_sub * G1) + sub * G1
            bt0 = boff // 128

            load_idx_tile(0, 0, boff, isem).wait()
            for t in range(RING):
                fire_gather(0, t, t)
            load_idx_tile(1, 1, boff, isem)

            @pl.loop(0, n_ht)
            def _(ht):
                tbt = ht & 1
                h0 = ht * HTILE
                for hr in range(HTILE):
                    s = hr & (RING - 1)
                    if hr == 0:
                        @pl.when(jnp.logical_and(ht >= 1, ht < n_ht - 1))
                        def _():
                            load_idx_tile(ht + 1, 1 - tbt, boff, isem)
                    wait_gather(s)
                    for bt in range(nbt):
                        s2 = bt & 1
                        first = (hr == 0) and (bt < 2)
                        if first:
                            @pl.when(ht > 0)
                            def _():
                                wait_store(h0 + hr, bt0 + bt, s2)
                        else:
                            wait_store(h0 + hr, bt0 + bt, s2)
                        transpose(s, bt, s2)
                        fire_store(h0 + hr, bt0 + bt, s2)
                    if hr < HTILE - RING:
                        fire_gather(tbt, hr + RING, s)
                    else:
                        if hr == HTILE - RING:
                            @pl.when(ht < n_ht - 1)
                            def _():
                                pltpu.make_async_copy(
                                    idx_hbm.at[pl.ds((ht + 1) * HTILE, HTILE),
                                               pl.ds(boff, G1)],
                                    idx_v.at[1 - tbt], isem).wait()

                        @pl.when(ht < n_ht - 1)
                        def _():
                            fire_gather(1 - tbt, hr - (HTILE - RING), s)

            wait_store(hist - 1, bt0 + nbt - 2, 0)
            wait_store(hist - 1, bt0 + nbt - 1, 1)

    return k


def kernel(input, table):
    batch, hist = input.shape
    vocab, embed = table.shape
    idx_t = jnp.transpose(input.astype(jnp.int32))
    out5 = _make_kernel(hist, batch, vocab, embed)(idx_t, table)
    return jnp.transpose(out5, (2, 4, 0, 1, 3)).reshape(batch, hist, embed)


# speedup vs baseline: 1.0093x; 1.0018x over previous
import functools

import jax
import jax.numpy as jnp
from jax import lax
from jax.experimental import pallas as pl
from jax.experimental.pallas import tpu as pltpu
from jax.experimental.pallas import tpu_sc as plsc

NUM_WORKERS = 32
HTILE = 8
G1 = 512
RING = 4


@functools.lru_cache(maxsize=None)
def _make_kernel(hist: int, batch: int, vocab: int, embed: int):
    assert batch % (NUM_WORKERS * G1) == 0 and hist % HTILE == 0
    assert embed % 8 == 0 and batch % 128 == 0
    n_sub = batch // (NUM_WORKERS * G1)
    n_ht = hist // HTILE
    nbt = G1 // 128
    mesh = plsc.VectorSubcoreMesh(core_axis_name="c", subcore_axis_name="s")

    @functools.partial(
        pl.kernel,
        mesh=mesh,
        out_type=jax.ShapeDtypeStruct(
            (hist, embed // 8, batch // 128, 8, 128), jnp.float32),
        scratch_types=[
            pltpu.VMEM((2, HTILE, G1), jnp.int32),
            [pltpu.VMEM((G1,), jnp.int32)] * RING,
            pltpu.VMEM((RING, G1, embed), jnp.float32),
            pltpu.VMEM((2, embed // 8, 8, 128), jnp.float32),
            [pltpu.SemaphoreType.DMA] * RING,
            [pltpu.SemaphoreType.DMA] * 2,
            pltpu.SemaphoreType.DMA,
        ],
        compiler_params=pltpu.CompilerParams(
            use_tc_tiling_on_sc=False, needs_layout_passes=False),
    )
    def k(idx_hbm, table_hbm, out_hbm, idx_v, j_v, rows_v, blk_v, gsems,
          ssems, isem):
        wid = lax.axis_index("s") * 2 + lax.axis_index("c")
        iota = lax.iota(jnp.int32, 16)
        zeros16 = iota >> 4

        def load_idx_tile(ht, tb, boff, sem):
            return pltpu.async_copy(
                idx_hbm.at[pl.ds(ht * HTILE, HTILE), pl.ds(boff, G1)],
                idx_v.at[tb], sem)

        def fire_gather(tb, hrow, s):
            for i in range(G1 // 16):
                v = idx_v[tb, hrow, pl.ds(i * 16, 16)]
                j_v[s][pl.ds(i * 16, 16)] = v
            pltpu.async_copy(table_hbm.at[j_v[s]], rows_v.at[s], gsems[s])

        def wait_gather(s):
            pltpu.make_async_copy(
                table_hbm.at[j_v[s]], rows_v.at[s], gsems[s]).wait()

        def transpose(s, bt, s2):
            @plsc.parallel_loop(0, embed, unroll=4)
            def _(e):
                for bg in range(8):
                    row = bt * 128 + bg * 16 + iota
                    g = plsc.load_gather(rows_v.at[s], [row, zeros16 + e])
                    blk_v[s2, e >> 3, e & 7, pl.ds(bg * 16, 16)] = g

        def fire_store(h, btg, s2):
            pltpu.async_copy(
                blk_v.at[s2], out_hbm.at[h, pl.ds(0, embed // 8), btg],
                ssems[s2])

        def wait_store(h, btg, s2):
            pltpu.make_async_copy(
                blk_v.at[s2], out_hbm.at[h, pl.ds(0, embed // 8), btg],
                ssems[s2]).wait()

        for sub in range(n_sub):
            boff = wid * (n_sub * G1) + sub * G1
            bt0 = boff // 128

            load_idx_tile(0, 0, boff, isem).wait()
            for t in range(RING):
                fire_gather(0, t, t)
            load_idx_tile(1, 1, boff, isem)

            @pl.loop(0, n_ht)
            def _(ht):
                tbt = ht & 1
                h0 = ht * HTILE
                for hr in range(HTILE):
                    s = hr & (RING - 1)
                    if hr == 0:
                        @pl.when(jnp.logical_and(ht >= 1, ht < n_ht - 1))
                        def _():
                            load_idx_tile(ht + 1, 1 - tbt, boff, isem)
                    wait_gather(s)
                    for bt in range(nbt):
                        s2 = bt & 1
                        first = (hr == 0) and (bt < 2)
                        if first:
                            @pl.when(ht > 0)
                            def _():
                                wait_store(h0 + hr, bt0 + bt, s2)
                        else:
                            wait_store(h0 + hr, bt0 + bt, s2)
                        transpose(s, bt, s2)
                        fire_store(h0 + hr, bt0 + bt, s2)
                    if hr < HTILE - RING:
                        fire_gather(tbt, hr + RING, s)
                    else:
                        if hr == HTILE - RING:
                            @pl.when(ht < n_ht - 1)
                            def _():
                                pltpu.make_async_copy(
                                    idx_hbm.at[pl.ds((ht + 1) * HTILE, HTILE),
                                               pl.ds(boff, G1)],
                                    idx_v.at[1 - tbt], isem).wait()

                        @pl.when(ht < n_ht - 1)
                        def _():
                            fire_gather(1 - tbt, hr - (HTILE - RING), s)

            wait_store(hist - 1, bt0 + nbt - 2, 0)
            wait_store(hist - 1, bt0 + nbt - 1, 1)

    return k


def kernel(input, table):
    batch, hist = input.shape
    vocab, embed = table.shape
    idx_t = jnp.transpose(input.astype(jnp.int32))
    out5 = _make_kernel(hist, batch, vocab, embed)(idx_t, table)
    return jnp.transpose(out5, (2, 4, 0, 1, 3)).reshape(batch, hist, embed)


# speedup vs baseline: 2.3120x; 2.2907x over previous
import functools

import jax
import jax.numpy as jnp
from jax import lax
from jax.experimental import pallas as pl
from jax.experimental.pallas import tpu as pltpu
from jax.experimental.pallas import tpu_sc as plsc

NUM_WORKERS = 32
HTILE = 8
G1 = 512
RING = 4


@functools.lru_cache(maxsize=None)
def _make_kernel(hist: int, batch: int, vocab: int, embed: int):
    assert batch % (NUM_WORKERS * G1) == 0 and hist % HTILE == 0
    assert embed % 8 == 0 and batch % 128 == 0
    n_sub = batch // (NUM_WORKERS * G1)
    n_ht = hist // HTILE
    nbt = G1 // 128
    mesh = plsc.VectorSubcoreMesh(core_axis_name="c", subcore_axis_name="s")

    @functools.partial(
        pl.kernel,
        mesh=mesh,
        out_type=jax.ShapeDtypeStruct(
            (hist, embed // 8, batch // 128, 8, 128), jnp.float32),
        scratch_types=[
            pltpu.VMEM((2, HTILE, G1), jnp.int32),
            [pltpu.VMEM((G1,), jnp.int32)] * RING,
            pltpu.VMEM((RING, G1, embed), jnp.float32),
            pltpu.VMEM((2, embed // 8, 8, 133), jnp.float32),
            [pltpu.SemaphoreType.DMA] * RING,
            [pltpu.SemaphoreType.DMA] * 2,
            pltpu.SemaphoreType.DMA,
        ],
        compiler_params=pltpu.CompilerParams(
            use_tc_tiling_on_sc=False, needs_layout_passes=False),
    )
    def k(idx_hbm, table_hbm, out_hbm, idx_v, j_v, rows_v, blk_v, gsems,
          ssems, isem):
        wid = lax.axis_index("s") * 2 + lax.axis_index("c")
        iota = lax.iota(jnp.int32, 16)
        zeros16 = iota >> 4

        def load_idx_tile(ht, tb, boff, sem):
            return pltpu.async_copy(
                idx_hbm.at[pl.ds(ht * HTILE, HTILE), pl.ds(boff, G1)],
                idx_v.at[tb], sem)

        def fire_gather(tb, hrow, s):
            for i in range(G1 // 16):
                v = idx_v[tb, hrow, pl.ds(i * 16, 16)]
                j_v[s][pl.ds(i * 16, 16)] = v
            pltpu.async_copy(table_hbm.at[j_v[s]], rows_v.at[s], gsems[s])

        def wait_gather(s):
            pltpu.make_async_copy(
                table_hbm.at[j_v[s]], rows_v.at[s], gsems[s]).wait()

        etv = [(iota >> 3) + 2 * half for half in range(embed // 16)]
        erv = iota & 7

        def transpose(s, bt, s2):
            @plsc.parallel_loop(0, 128, unroll=4)
            def _(bc):
                col = zeros16 + bc
                for half in range(embed // 16):
                    x = rows_v[s, bt * 128 + bc, pl.ds(half * 16, 16)]
                    plsc.store_scatter(blk_v.at[s2], [etv[half], erv, col], x)

        def fire_store(h, btg, s2):
            pltpu.async_copy(
                blk_v.at[s2, pl.ds(0, embed // 8), pl.ds(0, 8), pl.ds(0, 128)],
                out_hbm.at[h, pl.ds(0, embed // 8), btg], ssems[s2])

        def wait_store(h, btg, s2):
            pltpu.make_async_copy(
                blk_v.at[s2, pl.ds(0, embed // 8), pl.ds(0, 8), pl.ds(0, 128)],
                out_hbm.at[h, pl.ds(0, embed // 8), btg], ssems[s2]).wait()

        for sub in range(n_sub):
            boff = wid * (n_sub * G1) + sub * G1
            bt0 = boff // 128

            load_idx_tile(0, 0, boff, isem).wait()
            for t in range(RING):
                fire_gather(0, t, t)
            load_idx_tile(1, 1, boff, isem)

            @pl.loop(0, n_ht)
            def _(ht):
                tbt = ht & 1
                h0 = ht * HTILE
                for hr in range(HTILE):
                    s = hr & (RING - 1)
                    if hr == 0:
                        @pl.when(jnp.logical_and(ht >= 1, ht < n_ht - 1))
                        def _():
                            load_idx_tile(ht + 1, 1 - tbt, boff, isem)
                    wait_gather(s)
                    for bt in range(nbt):
                        s2 = bt & 1
                        first = (hr == 0) and (bt < 2)
                        if first:
                            @pl.when(ht > 0)
                            def _():
                                wait_store(h0 + hr, bt0 + bt, s2)
                        else:
                            wait_store(h0 + hr, bt0 + bt, s2)
                        transpose(s, bt, s2)
                        fire_store(h0 + hr, bt0 + bt, s2)
                    if hr < HTILE - RING:
                        fire_gather(tbt, hr + RING, s)
                    else:
                        if hr == HTILE - RING:
                            @pl.when(ht < n_ht - 1)
                            def _():
                                pltpu.make_async_copy(
                                    idx_hbm.at[pl.ds((ht + 1) * HTILE, HTILE),
                                               pl.ds(boff, G1)],
                                    idx_v.at[1 - tbt], isem).wait()

                        @pl.when(ht < n_ht - 1)
                        def _():
                            fire_gather(1 - tbt, hr - (HTILE - RING), s)

            wait_store(hist - 1, bt0 + nbt - 2, 0)
            wait_store(hist - 1, bt0 + nbt - 1, 1)

    return k


def kernel(input, table):
    batch, hist = input.shape
    vocab, embed = table.shape
    idx_t = jnp.transpose(input.astype(jnp.int32))
    out5 = _make_kernel(hist, batch, vocab, embed)(idx_t, table)
    return jnp.transpose(out5, (2, 4, 0, 1, 3)).reshape(batch, hist, embed)
